# baseline (speedup 1.0000x reference)
import functools

import jax
import jax.numpy as jnp
from jax import lax
from jax.experimental import pallas as pl
from jax.experimental.pallas import tpu as pltpu
from jax.experimental.pallas import tpu_sc as plsc

QUEUE_SIZE = 32768
BATCH = 4096
DIM = 64
NUM_CORES = 2
NUM_SUBCORES = 16
NUM_WORKERS = NUM_CORES * NUM_SUBCORES
ROW_G = 8
N_ROW_G = DIM // ROW_G
N_COL_S = NUM_WORKERS // N_ROW_G
STRIPE = QUEUE_SIZE // N_COL_S
CHUNK = BATCH
NCHUNKS = STRIPE // CHUNK


def _enqueue_body(key_t, queue_t, ptr_hbm, out_t, out_ptr,
                  ptr_v, buf0, buf1, kbuf, rsem0, rsem1, ksem, wsem):
    cid = lax.axis_index("c")
    sid = lax.axis_index("s")
    wid = cid * NUM_SUBCORES + sid
    r0 = pl.multiple_of((wid // N_COL_S) * ROW_G, ROW_G)
    cs0 = (wid % N_COL_S) * STRIPE
    bufs = (buf0, buf1)
    rsems = (rsem0, rsem1)

    for j in range(NCHUNKS):
        c0 = pl.multiple_of(cs0 + j * CHUNK, 128)
        pltpu.make_async_copy(
            queue_t.at[pl.ds(r0, ROW_G), pl.ds(c0, CHUNK)], bufs[j], rsems[j]
        ).start()
    pltpu.make_async_copy(key_t.at[pl.ds(r0, ROW_G)], kbuf, ksem).start()

    pltpu.sync_copy(ptr_hbm, ptr_v.at[pl.ds(0, 1)])
    praw = ptr_v[...][0]
    start = pl.multiple_of(jnp.clip(praw, 0, QUEUE_SIZE - BATCH), 128)

    for j in range(NCHUNKS):
        c0 = pl.multiple_of(cs0 + j * CHUNK, 128)
        pltpu.make_async_copy(
            queue_t.at[pl.ds(r0, ROW_G), pl.ds(c0, CHUNK)], bufs[j], rsems[j]
        ).wait()

        lo = jnp.maximum(c0, start)
        hi = jnp.minimum(c0 + CHUNK, start + BATCH)

        @pl.when(jnp.logical_and(c0 != start, lo < hi))
        def _():
            def piece(k, carry):
                col = lo + k * 128
                pltpu.sync_copy(
                    key_t.at[pl.ds(r0, ROW_G),
                             pl.ds(pl.multiple_of(col - start, 128), 128)],
                    bufs[j].at[:, pl.ds(pl.multiple_of(col - c0, 128), 128)],
                )
                return carry

            lax.fori_loop(0, (hi - lo) // 128, piece, 0)

        @pl.when(c0 != start)
        def _():
            pltpu.make_async_copy(
                bufs[j], out_t.at[pl.ds(r0, ROW_G), pl.ds(c0, CHUNK)], wsem
            ).start()

    pltpu.make_async_copy(key_t.at[pl.ds(r0, ROW_G)], kbuf, ksem).wait()
    for j in range(NCHUNKS):
        c0 = pl.multiple_of(cs0 + j * CHUNK, 128)

        @pl.when(c0 == start)
        def _():
            pltpu.make_async_copy(
                kbuf, out_t.at[pl.ds(r0, ROW_G), pl.ds(c0, CHUNK)], wsem
            ).start()

    @pl.when(wid == 0)
    def _():
        ptr_v[...] = jnp.full((16,), (praw + BATCH) % QUEUE_SIZE, jnp.int32)
        pltpu.sync_copy(ptr_v.at[pl.ds(0, 1)], out_ptr)

    for j in range(NCHUNKS):
        c0 = pl.multiple_of(cs0 + j * CHUNK, 128)
        pltpu.make_async_copy(
            bufs[j], out_t.at[pl.ds(r0, ROW_G), pl.ds(c0, CHUNK)], wsem
        ).wait()


_enqueue = functools.partial(
    pl.kernel,
    out_type=[
        jax.ShapeDtypeStruct((DIM, QUEUE_SIZE), jnp.float32),
        jax.ShapeDtypeStruct((1,), jnp.int32),
    ],
    scratch_types=[
        pltpu.VMEM((16,), jnp.int32),
        pltpu.VMEM((ROW_G, CHUNK), jnp.float32),
        pltpu.VMEM((ROW_G, CHUNK), jnp.float32),
        pltpu.VMEM((ROW_G, CHUNK), jnp.float32),
        pltpu.SemaphoreType.DMA,
        pltpu.SemaphoreType.DMA,
        pltpu.SemaphoreType.DMA,
        pltpu.SemaphoreType.DMA,
    ],
    mesh=plsc.VectorSubcoreMesh(core_axis_name="c", subcore_axis_name="s"),
)(_enqueue_body)


def kernel(key, queue, queue_ptr):
    out_t, new_ptr = _enqueue(key.T, queue.T, queue_ptr)
    return out_t.T, new_ptr


# speedup vs baseline: 1.0418x; 1.0418x over previous
import functools

import jax
import jax.numpy as jnp
from jax import lax
from jax.experimental import pallas as pl
from jax.experimental.pallas import tpu as pltpu
from jax.experimental.pallas import tpu_sc as plsc

QUEUE_SIZE = 32768
BATCH = 4096
DIM = 64
NUM_CORES = 2
NUM_SUBCORES = 16
NUM_WORKERS = NUM_CORES * NUM_SUBCORES
ROW_G = 8
N_ROW_G = DIM // ROW_G
N_COL_S = NUM_WORKERS // N_ROW_G
STRIPE = QUEUE_SIZE // N_COL_S
CHUNK = BATCH
NCHUNKS = STRIPE // CHUNK


def _enqueue_body(key_t, queue_t, ptr_hbm, out_t, out_ptr,
                  ptr_v, buf0, buf1, kbuf, rsem0, rsem1, ksem, wsem):
    cid = lax.axis_index("c")
    sid = lax.axis_index("s")
    wid = cid * NUM_SUBCORES + sid
    r0 = pl.multiple_of((wid // N_COL_S) * ROW_G, ROW_G)
    cs0 = (wid % N_COL_S) * STRIPE
    bufs = (buf0, buf1)
    rsems = (rsem0, rsem1)

    for j in range(NCHUNKS):
        c0 = pl.multiple_of(cs0 + j * CHUNK, 128)
        pltpu.make_async_copy(
            queue_t.at[pl.ds(r0, ROW_G), pl.ds(c0, CHUNK)], bufs[j], rsems[j]
        ).start()

    pltpu.sync_copy(ptr_hbm, ptr_v.at[pl.ds(0, 1)])
    praw = ptr_v[...][0]
    start = pl.multiple_of(jnp.clip(praw, 0, QUEUE_SIZE - BATCH), 128)

    for j in range(NCHUNKS):
        c0 = pl.multiple_of(cs0 + j * CHUNK, 128)

        @pl.when(c0 == start)
        def _():
            pltpu.make_async_copy(
                key_t.at[pl.ds(r0, ROW_G)], kbuf, ksem
            ).start()

    for j in range(NCHUNKS):
        c0 = pl.multiple_of(cs0 + j * CHUNK, 128)
        pltpu.make_async_copy(
            queue_t.at[pl.ds(r0, ROW_G), pl.ds(c0, CHUNK)], bufs[j], rsems[j]
        ).wait()

        lo = jnp.maximum(c0, start)
        hi = jnp.minimum(c0 + CHUNK, start + BATCH)

        @pl.when(jnp.logical_and(c0 != start, lo < hi))
        def _():
            def piece(k, carry):
                col = lo + k * 128
                pltpu.sync_copy(
                    key_t.at[pl.ds(r0, ROW_G),
                             pl.ds(pl.multiple_of(col - start, 128), 128)],
                    bufs[j].at[:, pl.ds(pl.multiple_of(col - c0, 128), 128)],
                )
                return carry

            lax.fori_loop(0, (hi - lo) // 128, piece, 0)

        @pl.when(c0 != start)
        def _():
            pltpu.make_async_copy(
                bufs[j], out_t.at[pl.ds(r0, ROW_G), pl.ds(c0, CHUNK)], wsem
            ).start()

    for j in range(NCHUNKS):
        c0 = pl.multiple_of(cs0 + j * CHUNK, 128)

        @pl.when(c0 == start)
        def _():
            pltpu.make_async_copy(
                key_t.at[pl.ds(r0, ROW_G)], kbuf, ksem
            ).wait()
            pltpu.make_async_copy(
                kbuf, out_t.at[pl.ds(r0, ROW_G), pl.ds(c0, CHUNK)], wsem
            ).start()

    @pl.when(wid == 0)
    def _():
        ptr_v[...] = jnp.full((16,), (praw + BATCH) % QUEUE_SIZE, jnp.int32)
        pltpu.sync_copy(ptr_v.at[pl.ds(0, 1)], out_ptr)

    for j in range(NCHUNKS):
        c0 = pl.multiple_of(cs0 + j * CHUNK, 128)
        pltpu.make_async_copy(
            bufs[j], out_t.at[pl.ds(r0, ROW_G), pl.ds(c0, CHUNK)], wsem
        ).wait()


_enqueue = functools.partial(
    pl.kernel,
    out_type=[
        jax.ShapeDtypeStruct((DIM, QUEUE_SIZE), jnp.float32),
        jax.ShapeDtypeStruct((1,), jnp.int32),
    ],
    scratch_types=[
        pltpu.VMEM((16,), jnp.int32),
        pltpu.VMEM((ROW_G, CHUNK), jnp.float32),
        pltpu.VMEM((ROW_G, CHUNK), jnp.float32),
        pltpu.VMEM((ROW_G, CHUNK), jnp.float32),
        pltpu.SemaphoreType.DMA,
        pltpu.SemaphoreType.DMA,
        pltpu.SemaphoreType.DMA,
        pltpu.SemaphoreType.DMA,
    ],
    mesh=plsc.VectorSubcoreMesh(core_axis_name="c", subcore_axis_name="s"),
)(_enqueue_body)


def kernel(key, queue, queue_ptr):
    out_t, new_ptr = _enqueue(key.T, queue.T, queue_ptr)
    return out_t.T, new_ptr


# speedup vs baseline: 1.0488x; 1.0067x over previous
import functools

import jax
import jax.numpy as jnp
from jax import lax
from jax.experimental import pallas as pl
from jax.experimental.pallas import tpu as pltpu
from jax.experimental.pallas import tpu_sc as plsc

QUEUE_SIZE = 32768
BATCH = 4096
DIM = 64
NUM_CORES = 2
NUM_SUBCORES = 16
NUM_WORKERS = NUM_CORES * NUM_SUBCORES
ROW_G = 8
N_ROW_G = DIM // ROW_G
N_COL_S = NUM_WORKERS // N_ROW_G
STRIPE = QUEUE_SIZE // N_COL_S
CHUNK = 2048
NCHUNKS = STRIPE // CHUNK


def _enqueue_body(key_t, queue_t, ptr_hbm, out_t, out_ptr, ptr_v, *scratch):
    bufs = scratch[:NCHUNKS]
    rsems = scratch[NCHUNKS:2 * NCHUNKS]
    wsem = scratch[2 * NCHUNKS]
    cid = lax.axis_index("c")
    sid = lax.axis_index("s")
    wid = cid * NUM_SUBCORES + sid
    r0 = pl.multiple_of((wid // N_COL_S) * ROW_G, ROW_G)
    cs0 = (wid % N_COL_S) * STRIPE

    pltpu.sync_copy(ptr_hbm, ptr_v.at[pl.ds(0, 1)])
    praw = ptr_v[...][0]
    start = pl.multiple_of(jnp.clip(praw, 0, QUEUE_SIZE - BATCH), 128)

    for j in range(NCHUNKS):
        c0 = pl.multiple_of(cs0 + j * CHUNK, 128)
        fully_in = jnp.logical_and(start <= c0, c0 + CHUNK <= start + BATCH)

        @pl.when(fully_in)
        def _():
            pltpu.make_async_copy(
                key_t.at[pl.ds(r0, ROW_G),
                         pl.ds(pl.multiple_of(c0 - start, 128), CHUNK)],
                bufs[j], rsems[j]
            ).start()

        @pl.when(jnp.logical_not(fully_in))
        def _():
            pltpu.make_async_copy(
                queue_t.at[pl.ds(r0, ROW_G), pl.ds(c0, CHUNK)], bufs[j],
                rsems[j]
            ).start()

    for j in range(NCHUNKS):
        c0 = pl.multiple_of(cs0 + j * CHUNK, 128)
        fully_in = jnp.logical_and(start <= c0, c0 + CHUNK <= start + BATCH)
        pltpu.make_async_copy(
            queue_t.at[pl.ds(r0, ROW_G), pl.ds(c0, CHUNK)], bufs[j], rsems[j]
        ).wait()

        lo = jnp.maximum(c0, start)
        hi = jnp.minimum(c0 + CHUNK, start + BATCH)

        @pl.when(jnp.logical_and(jnp.logical_not(fully_in), lo < hi))
        def _():
            def piece(k, carry):
                col = lo + k * 128
                pltpu.sync_copy(
                    key_t.at[pl.ds(r0, ROW_G),
                             pl.ds(pl.multiple_of(col - start, 128), 128)],
                    bufs[j].at[:, pl.ds(pl.multiple_of(col - c0, 128), 128)],
                )
                return carry

            lax.fori_loop(0, (hi - lo) // 128, piece, 0)

        pltpu.make_async_copy(
            bufs[j], out_t.at[pl.ds(r0, ROW_G), pl.ds(c0, CHUNK)], wsem
        ).start()

    @pl.when(wid == 0)
    def _():
        ptr_v[...] = jnp.full((16,), (praw + BATCH) % QUEUE_SIZE, jnp.int32)
        pltpu.sync_copy(ptr_v.at[pl.ds(0, 1)], out_ptr)

    for j in range(NCHUNKS):
        c0 = pl.multiple_of(cs0 + j * CHUNK, 128)
        pltpu.make_async_copy(
            bufs[j], out_t.at[pl.ds(r0, ROW_G), pl.ds(c0, CHUNK)], wsem
        ).wait()


_enqueue = functools.partial(
    pl.kernel,
    out_type=[
        jax.ShapeDtypeStruct((DIM, QUEUE_SIZE), jnp.float32),
        jax.ShapeDtypeStruct((1,), jnp.int32),
    ],
    scratch_types=(
        [pltpu.VMEM((16,), jnp.int32)]
        + [pltpu.VMEM((ROW_G, CHUNK), jnp.float32) for _ in range(NCHUNKS)]
        + [pltpu.SemaphoreType.DMA for _ in range(NCHUNKS)]
        + [pltpu.SemaphoreType.DMA]
    ),
    mesh=plsc.VectorSubcoreMesh(core_axis_name="c", subcore_axis_name="s"),
)(_enqueue_body)


def kernel(key, queue, queue_ptr):
    out_t, new_ptr = _enqueue(key.T, queue.T, queue_ptr)
    return out_t.T, new_ptr


# speedup vs baseline: 1.0636x; 1.0141x over previous
import functools

import jax
import jax.numpy as jnp
from jax import lax
from jax.experimental import pallas as pl
from jax.experimental.pallas import tpu as pltpu
from jax.experimental.pallas import tpu_sc as plsc

QUEUE_SIZE = 32768
BATCH = 4096
DIM = 64
NUM_CORES = 2
NUM_SUBCORES = 16
NUM_WORKERS = NUM_CORES * NUM_SUBCORES
ROW_G = 8
N_ROW_G = DIM // ROW_G
N_COL_S = NUM_WORKERS // N_ROW_G
STRIPE = QUEUE_SIZE // N_COL_S
CHUNK = 4096
NCHUNKS = STRIPE // CHUNK


def _enqueue_body(key_t, queue_t, ptr_hbm, out_t, out_ptr, ptr_v, *scratch):
    bufs = scratch[:NCHUNKS]
    rsems = scratch[NCHUNKS:2 * NCHUNKS]
    wsem = scratch[2 * NCHUNKS]
    cid = lax.axis_index("c")
    sid = lax.axis_index("s")
    wid = cid * NUM_SUBCORES + sid
    r0 = pl.multiple_of((wid // N_COL_S) * ROW_G, ROW_G)
    cs0 = (wid % N_COL_S) * STRIPE

    pltpu.sync_copy(ptr_hbm, ptr_v.at[pl.ds(0, 1)])
    praw = ptr_v[...][0]
    start = pl.multiple_of(jnp.clip(praw, 0, QUEUE_SIZE - BATCH), 128)

    for j in range(NCHUNKS):
        c0 = pl.multiple_of(cs0 + j * CHUNK, 128)
        fully_in = jnp.logical_and(start <= c0, c0 + CHUNK <= start + BATCH)

        @pl.when(fully_in)
        def _():
            pltpu.make_async_copy(
                key_t.at[pl.ds(r0, ROW_G),
                         pl.ds(pl.multiple_of(c0 - start, 128), CHUNK)],
                bufs[j], rsems[j]
            ).start()

        @pl.when(jnp.logical_not(fully_in))
        def _():
            pltpu.make_async_copy(
                queue_t.at[pl.ds(r0, ROW_G), pl.ds(c0, CHUNK)], bufs[j],
                rsems[j]
            ).start()

    for j in range(NCHUNKS):
        c0 = pl.multiple_of(cs0 + j * CHUNK, 128)
        fully_in = jnp.logical_and(start <= c0, c0 + CHUNK <= start + BATCH)
        pltpu.make_async_copy(
            queue_t.at[pl.ds(r0, ROW_G), pl.ds(c0, CHUNK)], bufs[j], rsems[j]
        ).wait()

        lo = jnp.maximum(c0, start)
        hi = jnp.minimum(c0 + CHUNK, start + BATCH)

        @pl.when(jnp.logical_and(jnp.logical_not(fully_in), lo < hi))
        def _():
            def piece(k, carry):
                col = lo + k * 128
                pltpu.sync_copy(
                    key_t.at[pl.ds(r0, ROW_G),
                             pl.ds(pl.multiple_of(col - start, 128), 128)],
                    bufs[j].at[:, pl.ds(pl.multiple_of(col - c0, 128), 128)],
                )
                return carry

            lax.fori_loop(0, (hi - lo) // 128, piece, 0)

        pltpu.make_async_copy(
            bufs[j], out_t.at[pl.ds(r0, ROW_G), pl.ds(c0, CHUNK)], wsem
        ).start()

    @pl.when(wid == 0)
    def _():
        ptr_v[...] = jnp.full((16,), (praw + BATCH) % QUEUE_SIZE, jnp.int32)
        pltpu.sync_copy(ptr_v.at[pl.ds(0, 1)], out_ptr)

    for j in range(NCHUNKS):
        c0 = pl.multiple_of(cs0 + j * CHUNK, 128)
        pltpu.make_async_copy(
            bufs[j], out_t.at[pl.ds(r0, ROW_G), pl.ds(c0, CHUNK)], wsem
        ).wait()


_enqueue = functools.partial(
    pl.kernel,
    out_type=[
        jax.ShapeDtypeStruct((DIM, QUEUE_SIZE), jnp.float32),
        jax.ShapeDtypeStruct((1,), jnp.int32),
    ],
    scratch_types=(
        [pltpu.VMEM((16,), jnp.int32)]
        + [pltpu.VMEM((ROW_G, CHUNK), jnp.float32) for _ in range(NCHUNKS)]
        + [pltpu.SemaphoreType.DMA for _ in range(NCHUNKS)]
        + [pltpu.SemaphoreType.DMA]
    ),
    mesh=plsc.VectorSubcoreMesh(core_axis_name="c", subcore_axis_name="s"),
)(_enqueue_body)


def kernel(key, queue, queue_ptr):
    out_t, new_ptr = _enqueue(key.T, queue.T, queue_ptr)
    return out_t.T, new_ptr
